# baseline (speedup 1.0000x reference)
import functools

import jax
import jax.numpy as jnp
from jax import lax
from jax.experimental import pallas as pl
from jax.experimental.pallas import tpu as pltpu
from jax.experimental.pallas import tpu_sc as plsc

V = 1000000
D = 64
B_TOK = 4096
S_TOK = 200
SCALE = 8.0
K1_W = 32768
K3_S = 8


def _k1_tc(t_t):
    def body(x_ref, o_ref):
        x = x_ref[...] * SCALE
        o_ref[...] = jnp.concatenate(
            [jnp.transpose(x), jnp.zeros((K1_W, D), jnp.float32)], axis=1)

    grid = (V + K1_W - 1) // K1_W
    return pl.pallas_call(
        body,
        grid=(grid,),
        in_specs=[pl.BlockSpec((D, K1_W), lambda i: (0, i))],
        out_specs=pl.BlockSpec((K1_W, 128), lambda i: (i, 0)),
        out_shape=jax.ShapeDtypeStruct((V, 128), jnp.float32),
    )(t_t)


def _k3_tc(out2):
    h = K3_S * B_TOK // 2

    def body(x_ref, o_ref):
        x = x_ref[...]
        for j in range(K3_S):
            xs = x[j * (B_TOK // 2):(j + 1) * (B_TOK // 2), :]
            o_ref[j, :, :] = jnp.concatenate(
                [jnp.transpose(xs[:, :D]), jnp.transpose(xs[:, D:])], axis=1)

    return pl.pallas_call(
        body,
        grid=(S_TOK // K3_S,),
        in_specs=[pl.BlockSpec((h, 128), lambda s: (s, 0))],
        out_specs=pl.BlockSpec((K3_S, D, B_TOK), lambda s: (s, 0, 0)),
        out_shape=jax.ShapeDtypeStruct((S_TOK, D, B_TOK), jnp.float32),
    )(out2)


def _make_k2():
    info = plsc.get_sparse_core_info()
    nc, ns = info.num_cores, info.num_subcores
    mesh = plsc.VectorSubcoreMesh(core_axis_name="c", subcore_axis_name="s")
    half = B_TOK // 2

    @functools.partial(
        pl.kernel,
        mesh=mesh,
        out_type=jax.ShapeDtypeStruct((S_TOK * half, 128), jnp.float32),
        scratch_types=[
            pltpu.VMEM((S_TOK, 128), jnp.int32),
            pltpu.VMEM((128, 128), jnp.float32),
            pltpu.VMEM((128, 128), jnp.float32),
            pltpu.VMEM((128, 128), jnp.float32),
            pltpu.VMEM((128, 128), jnp.float32),
            pltpu.SemaphoreType.DMA,
            pltpu.SemaphoreType.DMA,
            pltpu.SemaphoreType.DMA,
            pltpu.SemaphoreType.DMA,
            pltpu.SemaphoreType.DMA,
            pltpu.SemaphoreType.DMA,
            pltpu.SemaphoreType.DMA,
            pltpu.SemaphoreType.DMA,
        ],
        compiler_params=pltpu.CompilerParams(use_tc_tiling_on_sc=False),
    )
    def k2(t2p, sidx, out2, idx_v,
           g0, g1, g2, g3, gs0, gs1, gs2, gs3, ws0, ws1, ws2, ws3):
        w = lax.axis_index("s") * nc + lax.axis_index("c")
        pltpu.sync_copy(sidx.at[:, w], idx_v)
        row0 = 128 * (w % 16)
        col0 = D * (w // 16)
        bufs = (g0, g1, g2, g3)
        gsems = (gs0, gs1, gs2, gs3)
        wsems = (ws0, ws1, ws2, ws3)

        def fire_gather(s, j):
            pltpu.async_copy(t2p.at[idx_v.at[s]], bufs[j], gsems[j])

        def wait_gather(j):
            pltpu.make_async_copy(
                t2p.at[idx_v.at[0]], bufs[j], gsems[j]).wait()

        def out_slice(s):
            return out2.at[pl.ds(s * half + row0, 128), pl.ds(col0, D)]

        def fire_write(s, j):
            pltpu.async_copy(bufs[j].at[:, pl.ds(0, D)], out_slice(s), wsems[j])

        def wait_write(j):
            pltpu.make_async_copy(
                bufs[j].at[:, pl.ds(0, D)], out_slice(0), wsems[j]).wait()

        for j in range(3):
            fire_gather(j, j)

        @pl.loop(0, S_TOK, step=4)
        def _(a):
            for j in range(4):
                s = a + j
                jn = (j + 3) % 4
                wait_gather(j)
                fire_write(s, j)
                if j == 0:
                    @pl.when(a > 0)
                    def _():
                        wait_write(jn)
                else:
                    wait_write(jn)

                @pl.when(s + 3 < S_TOK)
                def _():
                    fire_gather(s + 3, jn)

        wait_write(3)

    return k2


def kernel(src, table):
    t_t = jnp.transpose(table)
    t2p = _k1_tc(t_t)
    sidx = jnp.transpose(src).astype(jnp.int32).reshape(S_TOK, 32, 128)
    out2 = _make_k2()(t2p, sidx)
    o3 = _k3_tc(out2)
    return jnp.transpose(o3, (2, 0, 1))
